# Initial kernel scaffold; baseline (speedup 1.0000x reference)
import jax
import jax.numpy as jnp
from jax.experimental import pallas as pl


def kernel(h, edge_index, edge_feat, weight, weight_edge, bias):
    raise NotImplementedError("write your pallas kernel here")


# speedup vs baseline: 3.4198x; 3.4198x over previous
import jax
import jax.numpy as jnp
from jax import lax
from jax.experimental import pallas as pl
from jax.experimental.pallas import tpu as pltpu
from jax.experimental.pallas import tpu_sc as plsc

N = 10000
E = 160000
F = 256
HF = 128
EFW = 32
EDGE_DIM = 16

NC = 2
NS = 16
K = 80
EPT = E // NS
CH = EPT // K
NP = 10240
RT = NP // NS
ZR = 32

E2 = 163840
EPW2 = E2 // (NC * NS)
K2 = 128
CH2 = EPW2 // K2


def _sc_h_body(hcat, src3, dst3, aggh_out,
               acc, srcb, dstb, idxg, rows, zb, sem):
    c = lax.axis_index("c")
    s = lax.axis_index("s")

    @pl.loop(0, ZR)
    def _zrow(r):
        zero16 = jnp.zeros((16,), jnp.float32)
        for j in range(HF // 16):
            zb[r, pl.ds(j * 16, 16)] = zero16

    @pl.loop(0, RT // ZR)
    def _zcp(t):
        pltpu.sync_copy(zb, acc.at[pl.ds(s * RT + t * ZR, ZR), :])

    plsc.subcore_barrier()

    pltpu.sync_copy(src3.at[s], srcb)
    pltpu.sync_copy(dst3.at[s], dstb)

    off = jnp.full((16,), c * N, jnp.int32)

    @pl.loop(0, CH)
    def _chunk(i):
        for j in range(K // 16):
            idxg[pl.ds(j * 16, 16)] = srcb[i, pl.ds(j * 16, 16)] + off
        pltpu.async_copy(hcat.at[idxg], rows, sem).wait()
        pltpu.sync_copy(rows, acc.at[dstb.at[i]], add=True)

    plsc.subcore_barrier()

    pltpu.sync_copy(acc.at[pl.ds(s * RT, RT), :],
                    aggh_out.at[c, pl.ds(s * RT, RT), :])


def _sc_e_body(dst3b, efp, agge_out, acc, dstb, pefb, rows128, zb, sem):
    c = lax.axis_index("c")
    s = lax.axis_index("s")
    w = c * NS + s

    @pl.loop(0, ZR)
    def _zrow(r):
        zero16 = jnp.zeros((16,), jnp.float32)
        for j in range(HF // 16):
            zb[r, pl.ds(j * 16, 16)] = zero16

    @pl.loop(0, RT // ZR)
    def _zcp(t):
        pltpu.sync_copy(zb, acc.at[pl.ds(s * RT + t * ZR, ZR), :])

    @pl.loop(0, K2)
    def _zrows(r):
        zero16 = jnp.zeros((16,), jnp.float32)
        for j in range(HF // 16):
            rows128[r, pl.ds(j * 16, 16)] = zero16

    plsc.subcore_barrier()

    pltpu.sync_copy(dst3b.at[w], dstb)

    @pl.loop(0, CH2)
    def _chunk(i):
        pltpu.sync_copy(efp.at[pl.ds(w * (EPW2 // 4) + i * (K2 // 4), K2 // 4), :],
                        pefb)
        for e in range(K2):
            q, r = e // 4, (e % 4) * EFW
            rows128[e, pl.ds(0, 16)] = pefb[q, pl.ds(r, 16)]
            rows128[e, pl.ds(16, 16)] = pefb[q, pl.ds(r + 16, 16)]
        pltpu.sync_copy(rows128, acc.at[dstb.at[i]], add=True)

    plsc.subcore_barrier()

    pltpu.sync_copy(acc.at[pl.ds(s * RT, RT), :],
                    agge_out.at[c, pl.ds(s * RT, RT), :])


@jax.jit
def _sc_aggregate(hcat, src3, dst3, dst3b, efp):
    mesh = plsc.VectorSubcoreMesh(core_axis_name="c", subcore_axis_name="s")
    aggh = pl.kernel(
        _sc_h_body,
        out_type=jax.ShapeDtypeStruct((NC, NP, HF), jnp.float32),
        mesh=mesh,
        scratch_types=(
            pltpu.VMEM_SHARED((NP, HF), jnp.float32),
            pltpu.VMEM((CH, K), jnp.int32),
            pltpu.VMEM((CH, K), jnp.int32),
            pltpu.VMEM((K,), jnp.int32),
            pltpu.VMEM((K, HF), jnp.float32),
            pltpu.VMEM((ZR, HF), jnp.float32),
            pltpu.SemaphoreType.DMA,
        ),
    )(hcat, src3, dst3)
    agge = pl.kernel(
        _sc_e_body,
        out_type=jax.ShapeDtypeStruct((NC, NP, HF), jnp.float32),
        mesh=mesh,
        scratch_types=(
            pltpu.VMEM_SHARED((NP, HF), jnp.float32),
            pltpu.VMEM((CH2, K2), jnp.int32),
            pltpu.VMEM((K2 // 4, HF), jnp.float32),
            pltpu.VMEM((K2, HF), jnp.float32),
            pltpu.VMEM((ZR, HF), jnp.float32),
            pltpu.SemaphoreType.DMA,
        ),
    )(dst3b, efp)
    return aggh, agge


def _tc_body(aggh_ref, agge_ref, w_ref, we_ref, b_ref, o_ref):
    a0 = aggh_ref[0]
    a1 = aggh_ref[1]
    e = agge_ref[0] + agge_ref[1]
    ef = e[:, :EDGE_DIM]
    deg = e[:, EDGE_DIM:EDGE_DIM + 1]
    wew = jnp.dot(we_ref[...], w_ref[...], preferred_element_type=jnp.float32)
    out = jnp.dot(a0, w_ref[:HF, :], preferred_element_type=jnp.float32)
    out += jnp.dot(a1, w_ref[HF:, :], preferred_element_type=jnp.float32)
    out += jnp.dot(ef, wew, preferred_element_type=jnp.float32)
    norm = lax.rsqrt(jnp.maximum(deg, 1.0))
    o_ref[...] = out * norm + b_ref[...]


@jax.jit
def _tc_finish(aggh, agge, weight, weight_edge, bias2d):
    blk = 1000
    return pl.pallas_call(
        _tc_body,
        grid=(N // blk,),
        in_specs=[
            pl.BlockSpec((NC, blk, HF), lambda i: (0, i, 0)),
            pl.BlockSpec((NC, blk, HF), lambda i: (0, i, 0)),
            pl.BlockSpec((F, F), lambda i: (0, 0)),
            pl.BlockSpec((EDGE_DIM, F), lambda i: (0, 0)),
            pl.BlockSpec((1, F), lambda i: (0, 0)),
        ],
        out_specs=pl.BlockSpec((blk, F), lambda i: (i, 0)),
        out_shape=jax.ShapeDtypeStruct((N, F), jnp.float32),
    )(aggh, agge, weight, weight_edge, bias2d)


def kernel(h, edge_index, edge_feat, weight, weight_edge, bias):
    hcat = jnp.concatenate([h[:, :HF], h[:, HF:]], axis=0)
    src3 = edge_index[0].reshape(NS, CH, K)
    dst3 = edge_index[1].reshape(NS, CH, K)
    ef = jnp.concatenate(
        [edge_feat,
         jnp.ones((E, 1), jnp.float32),
         jnp.zeros((E, EFW - EDGE_DIM - 1), jnp.float32)], axis=1)
    efp = jnp.concatenate(
        [ef, jnp.zeros((E2 - E, EFW), jnp.float32)]).reshape(E2 // 4, 4 * EFW)
    dst3b = jnp.concatenate(
        [edge_index[1], jnp.zeros((E2 - E,), jnp.int32)]).reshape(NC * NS, CH2, K2)

    aggh, agge = _sc_aggregate(hcat, src3, dst3, dst3b, efp)
    return _tc_finish(aggh, agge, weight, weight_edge, bias.reshape(1, F))


# speedup vs baseline: 3.7206x; 1.0880x over previous
import jax
import jax.numpy as jnp
from jax import lax
from jax.experimental import pallas as pl
from jax.experimental.pallas import tpu as pltpu
from jax.experimental.pallas import tpu_sc as plsc

N = 10000
E = 160000
F = 256
HF = 128
EFW = 32
EDGE_DIM = 16

NC = 2
NS = 16
K = 80
EPT = E // NS
CH = EPT // K
NP = 10240
RT = NP // NS
ZR = 32

E2 = 163840
EPW2 = E2 // (NC * NS)
K2 = 64
CH2 = EPW2 // K2


def _sc_h_body(hview, src3, dst3, aggh_out,
               acc, srcb, dstb, idxg, rows, sem):
    c = lax.axis_index("c")
    s = lax.axis_index("s")

    @pl.loop(0, K)
    def _zrow(r):
        zero16 = jnp.zeros((16,), jnp.float32)
        for j in range(HF // 16):
            rows[r, pl.ds(j * 16, 16)] = zero16

    @pl.loop(0, RT // K)
    def _zcp(t):
        pltpu.sync_copy(rows, acc.at[pl.ds(s * RT + t * K, K), :])

    plsc.subcore_barrier()

    pltpu.sync_copy(src3.at[s], srcb)
    pltpu.sync_copy(dst3.at[s], dstb)

    coff = jnp.full((16,), c, jnp.int32)

    @pl.loop(0, CH)
    def _chunk(i):
        for j in range(K // 16):
            v = srcb[i, pl.ds(j * 16, 16)]
            idxg[pl.ds(j * 16, 16)] = v + v + coff
        pltpu.async_copy(hview.at[idxg], rows, sem).wait()
        pltpu.sync_copy(rows, acc.at[dstb.at[i]], add=True)

    plsc.subcore_barrier()
    pltpu.sync_copy(acc.at[pl.ds(s * RT, RT), :],
                    aggh_out.at[c, pl.ds(s * RT, RT), :])


def _sc_e_body(dst3b, efp, agge_out,
               acc, dstb2, pefb, rows128, sem0, sem1):
    c = lax.axis_index("c")
    s = lax.axis_index("s")
    w = c * NS + s
    sems = (sem0, sem1)

    @pl.loop(0, K2)
    def _zrow(r):
        zero16 = jnp.zeros((16,), jnp.float32)
        for j in range(HF // 16):
            rows128[r, pl.ds(j * 16, 16)] = zero16

    @pl.loop(0, RT // K2)
    def _zcp(t):
        pltpu.sync_copy(rows128, acc.at[pl.ds(s * RT + t * K2, K2), :])

    plsc.subcore_barrier()

    pltpu.sync_copy(dst3b.at[w], dstb2)
    PR = K2 // 4
    base = w * (EPW2 // 4)
    pltpu.async_copy(efp.at[pl.ds(base, PR), :], pefb.at[0], sem0)
    pltpu.async_copy(efp.at[pl.ds(base + PR, PR), :], pefb.at[1], sem1)

    @pl.loop(0, CH2, step=2)
    def _chunk2(i):
        for b in range(2):
            ii = i + b
            pltpu.make_async_copy(efp.at[pl.ds(base + ii * PR, PR), :],
                                  pefb.at[b], sems[b]).wait()
            for e in range(K2):
                q, r = e // 4, (e % 4) * EFW
                rows128[e, pl.ds(0, 16)] = pefb[b, q, pl.ds(r, 16)]
                rows128[e, pl.ds(16, 16)] = pefb[b, q, pl.ds(r + 16, 16)]

            @pl.when(ii + 2 < CH2)
            def _(ii=ii, b=b):
                pltpu.async_copy(efp.at[pl.ds(base + (ii + 2) * PR, PR), :],
                                 pefb.at[b], sems[b])

            pltpu.sync_copy(rows128, acc.at[dstb2.at[ii]], add=True)

    plsc.subcore_barrier()
    pltpu.sync_copy(acc.at[pl.ds(s * RT, RT), :],
                    agge_out.at[c, pl.ds(s * RT, RT), :])


@jax.jit
def _sc_aggregate(hview, src3, dst3, dst3b, efp):
    mesh = plsc.VectorSubcoreMesh(core_axis_name="c", subcore_axis_name="s")
    aggh = pl.kernel(
        _sc_h_body,
        out_type=jax.ShapeDtypeStruct((NC, NP, HF), jnp.float32),
        mesh=mesh,
        scratch_types=(
            pltpu.VMEM_SHARED((NP, HF), jnp.float32),
            pltpu.VMEM((CH, K), jnp.int32),
            pltpu.VMEM((CH, K), jnp.int32),
            pltpu.VMEM((K,), jnp.int32),
            pltpu.VMEM((K, HF), jnp.float32),
            pltpu.SemaphoreType.DMA,
        ),
    )(hview, src3, dst3)
    agge = pl.kernel(
        _sc_e_body,
        out_type=jax.ShapeDtypeStruct((NC, NP, HF), jnp.float32),
        mesh=mesh,
        scratch_types=(
            pltpu.VMEM_SHARED((NP, HF), jnp.float32),
            pltpu.VMEM((CH2, K2), jnp.int32),
            pltpu.VMEM((2, K2 // 4, HF), jnp.float32),
            pltpu.VMEM((K2, HF), jnp.float32),
            pltpu.SemaphoreType.DMA,
            pltpu.SemaphoreType.DMA,
        ),
    )(dst3b, efp)
    return aggh, agge


def _tc_body(aggh_ref, agge_ref, w_ref, we_ref, b_ref, o_ref):
    a0 = aggh_ref[0]
    a1 = aggh_ref[1]
    e = agge_ref[0] + agge_ref[1]
    ef = e[:, :EDGE_DIM]
    deg = e[:, EDGE_DIM:EDGE_DIM + 1]
    wew = jnp.dot(we_ref[...], w_ref[...], preferred_element_type=jnp.float32)
    out = jnp.dot(a0, w_ref[:HF, :], preferred_element_type=jnp.float32)
    out += jnp.dot(a1, w_ref[HF:, :], preferred_element_type=jnp.float32)
    out += jnp.dot(ef, wew, preferred_element_type=jnp.float32)
    norm = lax.rsqrt(jnp.maximum(deg, 1.0))
    o_ref[...] = out * norm + b_ref[...]


@jax.jit
def _tc_finish(aggh, agge, weight, weight_edge, bias2d):
    blk = 1000
    return pl.pallas_call(
        _tc_body,
        grid=(N // blk,),
        in_specs=[
            pl.BlockSpec((NC, blk, HF), lambda i: (0, i, 0)),
            pl.BlockSpec((NC, blk, HF), lambda i: (0, i, 0)),
            pl.BlockSpec((F, F), lambda i: (0, 0)),
            pl.BlockSpec((EDGE_DIM, F), lambda i: (0, 0)),
            pl.BlockSpec((1, F), lambda i: (0, 0)),
        ],
        out_specs=pl.BlockSpec((blk, F), lambda i: (i, 0)),
        out_shape=jax.ShapeDtypeStruct((N, F), jnp.float32),
    )(aggh, agge, weight, weight_edge, bias2d)


def kernel(h, edge_index, edge_feat, weight, weight_edge, bias):
    hview = h.reshape(2 * N, HF)
    src3 = edge_index[0].reshape(NS, CH, K)
    dst3 = edge_index[1].reshape(NS, CH, K)
    ef = jnp.concatenate(
        [edge_feat,
         jnp.ones((E, 1), jnp.float32),
         jnp.zeros((E, EFW - EDGE_DIM - 1), jnp.float32)], axis=1)
    efp = jnp.concatenate(
        [ef, jnp.zeros((E2 - E, EFW), jnp.float32)]).reshape(E2 // 4, 4 * EFW)
    dst3b = jnp.concatenate(
        [edge_index[1], jnp.zeros((E2 - E,), jnp.int32)]).reshape(NC * NS, CH2, K2)

    aggh, agge = _sc_aggregate(hview, src3, dst3, dst3b, efp)
    return _tc_finish(aggh, agge, weight, weight_edge, bias.reshape(1, F))


# speedup vs baseline: 3.8763x; 1.0419x over previous
import jax
import jax.numpy as jnp
from jax import lax
from jax.experimental import pallas as pl
from jax.experimental.pallas import tpu as pltpu
from jax.experimental.pallas import tpu_sc as plsc

N = 10000
E = 160000
F = 256
HF = 128
EFW = 32
EDGE_DIM = 16

NC = 2
NS = 16
K = 80
EPT = E // NS
CH = EPT // K
NP = 10240
RT = NP // NS
ZR = 32

E2 = 163840
EPW2 = E2 // (NC * NS)
K2 = 64
CH2 = EPW2 // K2


def _sc_h_body(hview, src3, dst3, aggh_out,
               acc, srcb, dstb, idxg, rows, sem):
    c = lax.axis_index("c")
    s = lax.axis_index("s")

    @pl.loop(0, K)
    def _zrow(r):
        zero16 = jnp.zeros((16,), jnp.float32)
        for j in range(HF // 16):
            rows[r, pl.ds(j * 16, 16)] = zero16

    @pl.loop(0, RT // K)
    def _zcp(t):
        pltpu.sync_copy(rows, acc.at[pl.ds(s * RT + t * K, K), :])

    plsc.subcore_barrier()

    pltpu.sync_copy(src3.at[s], srcb)
    pltpu.sync_copy(dst3.at[s], dstb)

    coff = jnp.full((16,), c, jnp.int32)

    @pl.loop(0, CH)
    def _chunk(i):
        for j in range(K // 16):
            v = srcb[i, pl.ds(j * 16, 16)]
            idxg[pl.ds(j * 16, 16)] = v + v + coff
        pltpu.async_copy(hview.at[idxg], rows, sem).wait()
        pltpu.sync_copy(rows, acc.at[dstb.at[i]], add=True)

    plsc.subcore_barrier()
    pltpu.sync_copy(acc.at[pl.ds(s * RT, RT), :],
                    aggh_out.at[c, pl.ds(s * RT, RT), :])


def _sc_e_body(dst3b, efp, agge_out,
               acc, dstb2, pefb, rows128, sem0, sem1):
    c = lax.axis_index("c")
    s = lax.axis_index("s")
    w = c * NS + s
    sems = (sem0, sem1)

    @pl.loop(0, K2)
    def _zrow(r):
        zero16 = jnp.zeros((16,), jnp.float32)
        for j in range(HF // 16):
            rows128[r, pl.ds(j * 16, 16)] = zero16

    @pl.loop(0, RT // K2)
    def _zcp(t):
        pltpu.sync_copy(rows128, acc.at[pl.ds(s * RT + t * K2, K2), :])

    one_hot = jnp.where(lax.iota(jnp.int32, 16) == 0,
                        jnp.float32(1), jnp.float32(0))

    @pl.loop(0, K2)
    def _ones(r):
        rows128[r, pl.ds(EDGE_DIM, 16)] = one_hot

    plsc.subcore_barrier()

    pltpu.sync_copy(dst3b.at[w], dstb2)
    PR = K2 // 8
    base = w * (EPW2 // 8)
    pltpu.async_copy(efp.at[pl.ds(base, PR), :], pefb.at[0], sem0)
    pltpu.async_copy(efp.at[pl.ds(base + PR, PR), :], pefb.at[1], sem1)

    @pl.loop(0, CH2, step=2)
    def _chunk2(i):
        for b in range(2):
            ii = i + b
            pltpu.make_async_copy(efp.at[pl.ds(base + ii * PR, PR), :],
                                  pefb.at[b], sems[b]).wait()
            for e in range(K2):
                rows128[e, pl.ds(0, 16)] = pefb[b, e // 8,
                                                pl.ds((e % 8) * 16, 16)]

            @pl.when(ii + 2 < CH2)
            def _(ii=ii, b=b):
                pltpu.async_copy(efp.at[pl.ds(base + (ii + 2) * PR, PR), :],
                                 pefb.at[b], sems[b])

            pltpu.sync_copy(rows128, acc.at[dstb2.at[ii]], add=True)

    plsc.subcore_barrier()
    pltpu.sync_copy(acc.at[pl.ds(s * RT, RT), :],
                    agge_out.at[c, pl.ds(s * RT, RT), :])


@jax.jit
def _sc_aggregate(hview, src3, dst3, dst3b, efp):
    mesh = plsc.VectorSubcoreMesh(core_axis_name="c", subcore_axis_name="s")
    aggh = pl.kernel(
        _sc_h_body,
        out_type=jax.ShapeDtypeStruct((NC, NP, HF), jnp.float32),
        mesh=mesh,
        scratch_types=(
            pltpu.VMEM_SHARED((NP, HF), jnp.float32),
            pltpu.VMEM((CH, K), jnp.int32),
            pltpu.VMEM((CH, K), jnp.int32),
            pltpu.VMEM((K,), jnp.int32),
            pltpu.VMEM((K, HF), jnp.float32),
            pltpu.SemaphoreType.DMA,
        ),
    )(hview, src3, dst3)
    agge = pl.kernel(
        _sc_e_body,
        out_type=jax.ShapeDtypeStruct((NC, NP, HF), jnp.float32),
        mesh=mesh,
        scratch_types=(
            pltpu.VMEM_SHARED((NP, HF), jnp.float32),
            pltpu.VMEM((CH2, K2), jnp.int32),
            pltpu.VMEM((2, K2 // 8, HF), jnp.float32),
            pltpu.VMEM((K2, HF), jnp.float32),
            pltpu.SemaphoreType.DMA,
            pltpu.SemaphoreType.DMA,
        ),
    )(dst3b, efp)
    return aggh, agge


def _tc_body(aggh_ref, agge_ref, w_ref, we_ref, b_ref, o_ref):
    a0 = aggh_ref[0]
    a1 = aggh_ref[1]
    e = agge_ref[0] + agge_ref[1]
    ef = e[:, :EDGE_DIM]
    deg = e[:, EDGE_DIM:EDGE_DIM + 1]
    wew = jnp.dot(we_ref[...], w_ref[...], preferred_element_type=jnp.float32)
    out = jnp.dot(a0, w_ref[:HF, :], preferred_element_type=jnp.float32)
    out += jnp.dot(a1, w_ref[HF:, :], preferred_element_type=jnp.float32)
    out += jnp.dot(ef, wew, preferred_element_type=jnp.float32)
    norm = lax.rsqrt(jnp.maximum(deg, 1.0))
    o_ref[...] = out * norm + b_ref[...]


@jax.jit
def _tc_finish(aggh, agge, weight, weight_edge, bias2d):
    blk = 1000
    return pl.pallas_call(
        _tc_body,
        grid=(N // blk,),
        in_specs=[
            pl.BlockSpec((NC, blk, HF), lambda i: (0, i, 0)),
            pl.BlockSpec((NC, blk, HF), lambda i: (0, i, 0)),
            pl.BlockSpec((F, F), lambda i: (0, 0)),
            pl.BlockSpec((EDGE_DIM, F), lambda i: (0, 0)),
            pl.BlockSpec((1, F), lambda i: (0, 0)),
        ],
        out_specs=pl.BlockSpec((blk, F), lambda i: (i, 0)),
        out_shape=jax.ShapeDtypeStruct((N, F), jnp.float32),
    )(aggh, agge, weight, weight_edge, bias2d)


def kernel(h, edge_index, edge_feat, weight, weight_edge, bias):
    hview = h.reshape(2 * N, HF)
    src3 = edge_index[0].reshape(NS, CH, K)
    dst3 = edge_index[1].reshape(NS, CH, K)
    efp = jnp.concatenate(
        [edge_feat,
         jnp.zeros((E2 - E, EDGE_DIM), jnp.float32)]).reshape(E2 // 8, HF)
    dst3b = jnp.concatenate(
        [edge_index[1],
         jnp.full((E2 - E,), N, jnp.int32)]).reshape(NC * NS, CH2, K2)

    aggh, agge = _sc_aggregate(hview, src3, dst3, dst3b, efp)
    return _tc_finish(aggh, agge, weight, weight_edge, bias.reshape(1, F))


# speedup vs baseline: 3.8865x; 1.0026x over previous
import jax
import jax.numpy as jnp
from jax import lax
from jax.experimental import pallas as pl
from jax.experimental.pallas import tpu as pltpu
from jax.experimental.pallas import tpu_sc as plsc

N = 10000
E = 160000
F = 256
HF = 128
EFW = 32
EDGE_DIM = 16

NC = 2
NS = 16
K = 80
EPT = E // NS
CH = EPT // K
NP = 10240
RT = NP // NS
ZR = 32

E2 = 163840
EPW2 = E2 // (NC * NS)
K2 = 128
CH2 = EPW2 // K2


def _sc_h_body(hview, src3, dst3, aggh_out,
               acc, srcb, dstb, idxg, rows, sem):
    c = lax.axis_index("c")
    s = lax.axis_index("s")

    @pl.loop(0, K)
    def _zrow(r):
        zero16 = jnp.zeros((16,), jnp.float32)
        for j in range(HF // 16):
            rows[r, pl.ds(j * 16, 16)] = zero16

    @pl.loop(0, RT // K)
    def _zcp(t):
        pltpu.sync_copy(rows, acc.at[pl.ds(s * RT + t * K, K), :])

    plsc.subcore_barrier()

    pltpu.sync_copy(src3.at[s], srcb)
    pltpu.sync_copy(dst3.at[s], dstb)

    coff = jnp.full((16,), c, jnp.int32)

    @pl.loop(0, CH)
    def _chunk(i):
        for j in range(K // 16):
            v = srcb[i, pl.ds(j * 16, 16)]
            idxg[pl.ds(j * 16, 16)] = v + v + coff
        pltpu.async_copy(hview.at[idxg], rows, sem).wait()
        pltpu.sync_copy(rows, acc.at[dstb.at[i]], add=True)

    plsc.subcore_barrier()
    pltpu.sync_copy(acc.at[pl.ds(s * RT, RT), :],
                    aggh_out.at[c, pl.ds(s * RT, RT), :])


def _sc_e_body(dst3b, efp, agge_out,
               acc, dstb2, pefb, rows128, sem0, sem1):
    c = lax.axis_index("c")
    s = lax.axis_index("s")
    w = c * NS + s
    sems = (sem0, sem1)

    @pl.loop(0, K2)
    def _zrow(r):
        zero16 = jnp.zeros((16,), jnp.float32)
        for j in range(HF // 16):
            rows128[r, pl.ds(j * 16, 16)] = zero16

    @pl.loop(0, RT // K2)
    def _zcp(t):
        pltpu.sync_copy(rows128, acc.at[pl.ds(s * RT + t * K2, K2), :])

    one_hot = jnp.where(lax.iota(jnp.int32, 16) == 0,
                        jnp.float32(1), jnp.float32(0))

    @pl.loop(0, K2)
    def _ones(r):
        rows128[r, pl.ds(EDGE_DIM, 16)] = one_hot

    plsc.subcore_barrier()

    pltpu.sync_copy(dst3b.at[w], dstb2)
    PR = K2 // 8
    base = w * (EPW2 // 8)
    pltpu.async_copy(efp.at[pl.ds(base, PR), :], pefb.at[0], sem0)
    pltpu.async_copy(efp.at[pl.ds(base + PR, PR), :], pefb.at[1], sem1)

    @pl.loop(0, CH2, step=2)
    def _chunk2(i):
        for b in range(2):
            ii = i + b
            pltpu.make_async_copy(efp.at[pl.ds(base + ii * PR, PR), :],
                                  pefb.at[b], sems[b]).wait()
            for e in range(K2):
                rows128[e, pl.ds(0, 16)] = pefb[b, e // 8,
                                                pl.ds((e % 8) * 16, 16)]

            @pl.when(ii + 2 < CH2)
            def _(ii=ii, b=b):
                pltpu.async_copy(efp.at[pl.ds(base + (ii + 2) * PR, PR), :],
                                 pefb.at[b], sems[b])

            pltpu.sync_copy(rows128, acc.at[dstb2.at[ii]], add=True)

    plsc.subcore_barrier()
    pltpu.sync_copy(acc.at[pl.ds(s * RT, RT), :],
                    agge_out.at[c, pl.ds(s * RT, RT), :])


@jax.jit
def _sc_aggregate(hview, src3, dst3, dst3b, efp):
    mesh = plsc.VectorSubcoreMesh(core_axis_name="c", subcore_axis_name="s")
    aggh = pl.kernel(
        _sc_h_body,
        out_type=jax.ShapeDtypeStruct((NC, NP, HF), jnp.float32),
        mesh=mesh,
        scratch_types=(
            pltpu.VMEM_SHARED((NP, HF), jnp.float32),
            pltpu.VMEM((CH, K), jnp.int32),
            pltpu.VMEM((CH, K), jnp.int32),
            pltpu.VMEM((K,), jnp.int32),
            pltpu.VMEM((K, HF), jnp.float32),
            pltpu.SemaphoreType.DMA,
        ),
    )(hview, src3, dst3)
    agge = pl.kernel(
        _sc_e_body,
        out_type=jax.ShapeDtypeStruct((NC, NP, HF), jnp.float32),
        mesh=mesh,
        scratch_types=(
            pltpu.VMEM_SHARED((NP, HF), jnp.float32),
            pltpu.VMEM((CH2, K2), jnp.int32),
            pltpu.VMEM((2, K2 // 8, HF), jnp.float32),
            pltpu.VMEM((K2, HF), jnp.float32),
            pltpu.SemaphoreType.DMA,
            pltpu.SemaphoreType.DMA,
        ),
    )(dst3b, efp)
    return aggh, agge


def _tc_body(aggh_ref, agge_ref, w_ref, we_ref, b_ref, o_ref):
    a0 = aggh_ref[0]
    a1 = aggh_ref[1]
    e = agge_ref[0] + agge_ref[1]
    ef = e[:, :EDGE_DIM]
    deg = e[:, EDGE_DIM:EDGE_DIM + 1]
    wew = jnp.dot(we_ref[...], w_ref[...], preferred_element_type=jnp.float32)
    out = jnp.dot(a0, w_ref[:HF, :], preferred_element_type=jnp.float32)
    out += jnp.dot(a1, w_ref[HF:, :], preferred_element_type=jnp.float32)
    out += jnp.dot(ef, wew, preferred_element_type=jnp.float32)
    norm = lax.rsqrt(jnp.maximum(deg, 1.0))
    o_ref[...] = out * norm + b_ref[...]


@jax.jit
def _tc_finish(aggh, agge, weight, weight_edge, bias2d):
    blk = 1000
    return pl.pallas_call(
        _tc_body,
        grid=(N // blk,),
        in_specs=[
            pl.BlockSpec((NC, blk, HF), lambda i: (0, i, 0)),
            pl.BlockSpec((NC, blk, HF), lambda i: (0, i, 0)),
            pl.BlockSpec((F, F), lambda i: (0, 0)),
            pl.BlockSpec((EDGE_DIM, F), lambda i: (0, 0)),
            pl.BlockSpec((1, F), lambda i: (0, 0)),
        ],
        out_specs=pl.BlockSpec((blk, F), lambda i: (i, 0)),
        out_shape=jax.ShapeDtypeStruct((N, F), jnp.float32),
    )(aggh, agge, weight, weight_edge, bias2d)


def kernel(h, edge_index, edge_feat, weight, weight_edge, bias):
    hview = h.reshape(2 * N, HF)
    src3 = edge_index[0].reshape(NS, CH, K)
    dst3 = edge_index[1].reshape(NS, CH, K)
    efp = jnp.concatenate(
        [edge_feat,
         jnp.zeros((E2 - E, EDGE_DIM), jnp.float32)]).reshape(E2 // 8, HF)
    dst3b = jnp.concatenate(
        [edge_index[1],
         jnp.full((E2 - E,), N, jnp.int32)]).reshape(NC * NS, CH2, K2)

    aggh, agge = _sc_aggregate(hview, src3, dst3, dst3b, efp)
    return _tc_finish(aggh, agge, weight, weight_edge, bias.reshape(1, F))
